# Initial kernel scaffold; baseline (speedup 1.0000x reference)
import jax
import jax.numpy as jnp
from jax.experimental import pallas as pl


def kernel(feats, batch_ids, W1, b1, W2, b2):
    raise NotImplementedError("write your pallas kernel here")


# speedup vs baseline: 2.7501x; 2.7501x over previous
import functools

import jax
import jax.numpy as jnp
from jax.experimental import pallas as pl
from jax.experimental.pallas import tpu as pltpu

N_TOK = 32768
N_BATCH = 16
IN_DIM = 4
HIDDEN = 64
FEAT_DIM = 16

GRID = 16
TILE = N_TOK // GRID


def _fused_body(feats_ref, ids_ref, w1_ref, b1_ref, w2_ref, b2_ref,
                out_ref, acc_ref, cnt_ref):
    pid = pl.program_id(0)

    x = feats_ref[...]
    h = jnp.maximum(
        jax.lax.dot_general(x, w1_ref[...], (((1,), (0,)), ((), ())),
                            preferred_element_type=jnp.float32)
        + b1_ref[...], 0.0)
    h = jax.lax.dot_general(h, w2_ref[...], (((1,), (0,)), ((), ())),
                            preferred_element_type=jnp.float32) + b2_ref[...]

    ids_row = ids_ref[0]
    iota_b = jax.lax.broadcasted_iota(jnp.int32, (N_BATCH, TILE), 0)
    mask_t = (iota_b == jnp.broadcast_to(ids_row, (N_BATCH, TILE)))
    cnt = jnp.sum(mask_t.astype(jnp.float32), axis=1, keepdims=True)

    ids_col = jnp.reshape(ids_row, (TILE, 1))
    neg_inf = jnp.float32(-jnp.inf)
    rows = []
    for b in range(N_BATCH):
        mb = jnp.max(jnp.where(ids_col == b, h, neg_inf), axis=0,
                     keepdims=True)
        rows.append(mb)
    part = jnp.concatenate(rows, axis=0)

    @pl.when(pid == 0)
    def _init():
        acc_ref[...] = jnp.full((N_BATCH, FEAT_DIM), neg_inf, jnp.float32)
        cnt_ref[...] = jnp.zeros((N_BATCH, 1), jnp.float32)

    acc_ref[...] = jnp.maximum(acc_ref[...], part)
    cnt_ref[...] = cnt_ref[...] + cnt

    @pl.when(pid == GRID - 1)
    def _finalize():
        acc = acc_ref[...]
        c = cnt_ref[...]
        padded = c < jnp.max(c)
        acc = jnp.where(padded, jnp.maximum(acc, 0.0), acc)
        norm = jnp.sqrt(jnp.sum(acc * acc, axis=1, keepdims=True))
        out_ref[...] = acc / jnp.maximum(norm, 1e-12)


@jax.jit
def kernel(feats, batch_ids, W1, b1, W2, b2):
    ids3 = batch_ids.reshape(GRID, 1, TILE)
    b1r = b1.reshape(1, HIDDEN)
    b2r = b2.reshape(1, FEAT_DIM)
    out = pl.pallas_call(
        _fused_body,
        grid=(GRID,),
        in_specs=[
            pl.BlockSpec((TILE, IN_DIM), lambda i: (i, 0)),
            pl.BlockSpec((1, 1, TILE), lambda i: (i, 0, 0)),
            pl.BlockSpec((IN_DIM, HIDDEN), lambda i: (0, 0)),
            pl.BlockSpec((1, HIDDEN), lambda i: (0, 0)),
            pl.BlockSpec((HIDDEN, FEAT_DIM), lambda i: (0, 0)),
            pl.BlockSpec((1, FEAT_DIM), lambda i: (0, 0)),
        ],
        out_specs=pl.BlockSpec((N_BATCH, FEAT_DIM), lambda i: (0, 0)),
        out_shape=jax.ShapeDtypeStruct((N_BATCH, FEAT_DIM), jnp.float32),
        scratch_shapes=[
            pltpu.VMEM((N_BATCH, FEAT_DIM), jnp.float32),
            pltpu.VMEM((N_BATCH, 1), jnp.float32),
        ],
    )(feats, ids3, W1, b1r, W2, b2r)
    return out
